# baseline (speedup 1.0000x reference)
import functools

import jax
import jax.numpy as jnp
from jax import lax
from jax.experimental import pallas as pl
from jax.experimental.pallas import tpu as pltpu
from jax.experimental.pallas import tpu_sc as plsc

_B, _L, _D, _EDIM, _VOCAB, _FR = 16, 2048, 256, 64, 50, 128
_N = _B * _L
_DW = _D // 2
_NW = 32
_RPW = _N // _NW
_C = 128
_NCH = _RPW // _C


def _leaky(x):
    return jnp.where(x >= 0, x, 0.01 * x)


def _stage1_body(h_ref, head_ref, tp_ref, W1_ref, b1_ref, Wc_ref, emb_ref,
                 We_ref, A_ref, SE_ref, idx_ref):
    b = pl.program_id(0)
    h = h_ref[0].astype(jnp.bfloat16)
    x = jnp.dot(h, W1_ref[:].astype(jnp.bfloat16),
                preferred_element_type=jnp.float32) + b1_ref[:]
    xg = _leaky(x).astype(jnp.bfloat16)
    AS = jnp.dot(xg, Wc_ref[:].astype(jnp.bfloat16),
                 preferred_element_type=jnp.float32)
    A_ref[0] = AS[:, :_D].astype(jnp.bfloat16)
    T = jnp.dot(emb_ref[:], We_ref[:], preferred_element_type=jnp.float32)
    tp = tp_ref[0, 0]
    ohe = (tp[:, None] == jax.lax.broadcasted_iota(jnp.int32, (_L, _VOCAB), 1)
           ).astype(jnp.float32)
    SE_ref[0] = (AS[:, _D:] + jnp.dot(ohe, T, preferred_element_type=jnp.float32)
                 ).astype(jnp.bfloat16)
    idx_ref[0] = head_ref[0] + b * _L


def _run_stage1(h, head, tp, W1, b1, Wc, emb, We):
    head3 = head.astype(jnp.int32).reshape(_B, 1, _L)
    tp3 = tp.astype(jnp.int32).reshape(_B, 1, _L)
    full = lambda *s: pl.BlockSpec(s, lambda b: (0,) * len(s))
    A, SE, idx = pl.pallas_call(
        _stage1_body,
        grid=(_B,),
        in_specs=[
            pl.BlockSpec((1, _L, _D), lambda b: (b, 0, 0)),
            pl.BlockSpec((1, 1, _L), lambda b: (b, 0, 0)),
            pl.BlockSpec((1, 1, _L), lambda b: (b, 0, 0)),
            full(_D, _D),
            full(1, _D),
            full(_D, 2 * _D),
            full(_VOCAB, _EDIM),
            full(_EDIM, _D),
        ],
        out_specs=[
            pl.BlockSpec((1, _L, _D), lambda b: (b, 0, 0)),
            pl.BlockSpec((1, _L, _D), lambda b: (b, 0, 0)),
            pl.BlockSpec((1, 1, _L), lambda b: (b, 0, 0)),
        ],
        out_shape=[
            jax.ShapeDtypeStruct((_B, _L, _D), jnp.bfloat16),
            jax.ShapeDtypeStruct((_B, _L, _D), jnp.bfloat16),
            jax.ShapeDtypeStruct((_B, 1, _L), jnp.int32),
        ],
        compiler_params=pltpu.CompilerParams(
            dimension_semantics=("arbitrary",)),
    )(h, head3, tp3, W1, b1.reshape(1, _D), Wc, emb, We)
    A32 = lax.bitcast_convert_type(A.reshape(_N, _DW, 2), jnp.int32)
    return A32, SE, idx.reshape(_N)


def _sc_gather(Ax, idxx, Ay, idxy):
    mesh = plsc.VectorSubcoreMesh(core_axis_name="c", subcore_axis_name="s")

    @functools.partial(
        pl.kernel,
        mesh=mesh,
        out_type=[
            jax.ShapeDtypeStruct((_N, _DW), jnp.int32),
            jax.ShapeDtypeStruct((_N, _DW), jnp.int32),
        ],
        scratch_types=[
            pltpu.VMEM((_C,), jnp.int32),
            pltpu.VMEM((_C, _DW), jnp.int32),
            pltpu.SemaphoreType.DMA,
        ],
        compiler_params=pltpu.CompilerParams(use_tc_tiling_on_sc=True),
    )
    def run(Ax_h, ix_h, Ay_h, iy_h, gx_h, gy_h, idxv, gv, sem):
        cid = lax.axis_index("c")
        sid = lax.axis_index("s")
        wid = sid * 2 + cid
        base = wid * _RPW
        for A_h, i_h, g_h in ((Ax_h, ix_h, gx_h), (Ay_h, iy_h, gy_h)):
            def chunk(i, carry):
                off = base + i * _C
                pltpu.sync_copy(i_h.at[pl.ds(off, _C)], idxv)
                pltpu.async_copy(A_h.at[idxv], gv, sem).wait()
                pltpu.sync_copy(gv, g_h.at[pl.ds(off, _C)])
                return carry
            lax.fori_loop(0, _NCH, chunk, 0)

    return run(Ax, idxx, Ay, idxy)


def _stage3_body(G_ref, SE_ref, frW_ref, frb_ref, out_ref):
    O = G_ref[0].astype(jnp.float32) + SE_ref[0].astype(jnp.float32)
    ss = jnp.sum(O * O, axis=1, keepdims=True)
    scale = 1.0 / jnp.maximum(jnp.sqrt(ss), 1e-12)
    rep = jnp.sum(O * scale, axis=0, keepdims=True) * (1.0 / _L)
    r = jnp.dot(rep, frW_ref[:], preferred_element_type=jnp.float32) + frb_ref[:]
    out_ref[0] = _leaky(r)


def _run_stage3(G, SE, frW, frb):
    full = lambda *s: pl.BlockSpec(s, lambda b: (0,) * len(s))
    out = pl.pallas_call(
        _stage3_body,
        grid=(_B,),
        in_specs=[
            pl.BlockSpec((1, _L, _D), lambda b: (b, 0, 0)),
            pl.BlockSpec((1, _L, _D), lambda b: (b, 0, 0)),
            full(_D, _FR),
            full(1, _FR),
        ],
        out_specs=pl.BlockSpec((1, 1, _FR), lambda b: (b, 0, 0)),
        out_shape=jax.ShapeDtypeStruct((_B, 1, _FR), jnp.float32),
        compiler_params=pltpu.CompilerParams(
            dimension_semantics=("arbitrary",)),
    )(G, SE, frW, frb.reshape(1, _FR))
    return out


def kernel(h_x, x_mask, src_token_dense_mask, src_token_sparse_mask,
           src_token_inarc_type, src_token_arc_head, src_token_depth,
           src_word_inarc_type, src_word_inarc_type_mask, h_y, y_mask,
           tgt_token_dense_mask, tgt_token_sparse_mask, tgt_token_inarc_type,
           tgt_token_arc_head, tgt_token_depth, tgt_word_inarc_type,
           tgt_word_inarc_type_mask, src_W, src_b, tgt_W, tgt_b, inarc_emb,
           W_msg, W_edge, W_skip, fr_W, fr_b):
    Wc = jnp.concatenate([W_msg, W_skip], axis=1)
    src_tp = src_word_inarc_type * src_word_inarc_type_mask
    tgt_tp = tgt_word_inarc_type * tgt_word_inarc_type_mask
    Ax, SEx, idxx = _run_stage1(h_x, src_token_arc_head, src_tp, src_W,
                                src_b, Wc, inarc_emb, W_edge)
    Ay, SEy, idxy = _run_stage1(h_y, tgt_token_arc_head, tgt_tp, tgt_W,
                                tgt_b, Wc, inarc_emb, W_edge)
    Gx32, Gy32 = _sc_gather(Ax, idxx, Ay, idxy)
    to_bf = lambda g: lax.bitcast_convert_type(
        g, jnp.bfloat16).reshape(_B, _L, _D)
    ox = _run_stage3(to_bf(Gx32), SEx, fr_W, fr_b)
    oy = _run_stage3(to_bf(Gy32), SEy, fr_W, fr_b)
    return (ox, oy)


# speedup vs baseline: 5.2543x; 5.2543x over previous
import functools

import jax
import jax.numpy as jnp
from jax.experimental import pallas as pl
from jax.experimental.pallas import tpu as pltpu

_B, _L, _D, _EDIM, _VOCAB, _FR = 16, 2048, 256, 64, 50, 128
_TL = 512


def _leaky(x):
    return jnp.where(x >= 0, x, 0.01 * x)


def _side_body(h_ref, head_ref, tp_ref, W1_ref, b1_ref, Wc_ref, emb_ref,
               We_ref, frW_ref, frb_ref, out_ref):
    h = h_ref[0].astype(jnp.bfloat16)
    x = jnp.dot(h, W1_ref[:].astype(jnp.bfloat16),
                preferred_element_type=jnp.float32) + b1_ref[:]
    xg = _leaky(x).astype(jnp.bfloat16)
    AS = jnp.dot(xg, Wc_ref[:].astype(jnp.bfloat16),
                 preferred_element_type=jnp.float32)
    A = AS[:, :_D].astype(jnp.bfloat16)
    S = AS[:, _D:]
    T = jnp.dot(emb_ref[:], We_ref[:], preferred_element_type=jnp.float32)
    tp = tp_ref[0, 0]
    ohe = (tp[:, None] == jax.lax.broadcasted_iota(jnp.int32, (_L, _VOCAB), 1)
           ).astype(jnp.float32)
    SE = S + jnp.dot(ohe, T, preferred_element_type=jnp.float32)
    head = head_ref[0, 0]
    col_iota = jax.lax.broadcasted_iota(jnp.int32, (_TL, _L), 1)

    acc = jnp.zeros((1, _D), jnp.float32)
    for i in range(_L // _TL):
        hd = head[i * _TL:(i + 1) * _TL]
        oh = (hd[:, None] == col_iota).astype(jnp.bfloat16)
        g = jnp.dot(oh, A, preferred_element_type=jnp.float32)
        o = g + SE[i * _TL:(i + 1) * _TL, :]
        ss = jnp.sum(o * o, axis=1, keepdims=True)
        scale = jax.lax.rsqrt(jnp.maximum(ss, 1e-24))
        acc = acc + jnp.sum(o * scale, axis=0, keepdims=True)
    rep = acc * (1.0 / _L)
    r = jnp.dot(rep, frW_ref[:], preferred_element_type=jnp.float32) + frb_ref[:]
    out_ref[0] = _leaky(r)


def _run_side(h, head, tp, W1, b1, Wc, emb, We, frW, frb, interpret=False):
    head3 = head.astype(jnp.int32).reshape(_B, 1, _L)
    tp3 = tp.astype(jnp.int32).reshape(_B, 1, _L)
    full = lambda *s: pl.BlockSpec(s, lambda b: (0,) * len(s))
    return pl.pallas_call(
        _side_body,
        grid=(_B,),
        in_specs=[
            pl.BlockSpec((1, _L, _D), lambda b: (b, 0, 0)),
            pl.BlockSpec((1, 1, _L), lambda b: (b, 0, 0)),
            pl.BlockSpec((1, 1, _L), lambda b: (b, 0, 0)),
            full(_D, _D),
            full(1, _D),
            full(_D, 2 * _D),
            full(_VOCAB, _EDIM),
            full(_EDIM, _D),
            full(_D, _FR),
            full(1, _FR),
        ],
        out_specs=pl.BlockSpec((1, 1, _FR), lambda b: (b, 0, 0)),
        out_shape=jax.ShapeDtypeStruct((_B, 1, _FR), jnp.float32),
        compiler_params=pltpu.CompilerParams(
            dimension_semantics=("arbitrary",)),
        interpret=interpret,
    )(h, head3, tp3, W1, b1.reshape(1, _D), Wc, emb, We, frW,
      frb.reshape(1, _FR))


def kernel(h_x, x_mask, src_token_dense_mask, src_token_sparse_mask,
           src_token_inarc_type, src_token_arc_head, src_token_depth,
           src_word_inarc_type, src_word_inarc_type_mask, h_y, y_mask,
           tgt_token_dense_mask, tgt_token_sparse_mask, tgt_token_inarc_type,
           tgt_token_arc_head, tgt_token_depth, tgt_word_inarc_type,
           tgt_word_inarc_type_mask, src_W, src_b, tgt_W, tgt_b, inarc_emb,
           W_msg, W_edge, W_skip, fr_W, fr_b, interpret=False):
    Wc = jnp.concatenate([W_msg, W_skip], axis=1)
    src_tp = src_word_inarc_type * src_word_inarc_type_mask
    tgt_tp = tgt_word_inarc_type * tgt_word_inarc_type_mask
    src_repr = _run_side(h_x, src_token_arc_head, src_tp, src_W, src_b, Wc,
                         inarc_emb, W_edge, fr_W, fr_b, interpret=interpret)
    y_repr = _run_side(h_y, tgt_token_arc_head, tgt_tp, tgt_W, tgt_b, Wc,
                       inarc_emb, W_edge, fr_W, fr_b, interpret=interpret)
    return (src_repr, y_repr)
